# baseline (device time: 49445 ns/iter reference)
import functools

import jax
import jax.numpy as jnp
from jax import lax
from jax.experimental import pallas as pl
from jax.experimental.pallas import tpu as pltpu

N_DEV = 8
B = 2
SQ = 256
SKV = 256
HQ_LOCAL = 4
DH = 64
DM = 512
SCALE = 0.125


def kernel(x, Wq, K_ext, V_ext, Wo):
    my = lax.axis_index("i")
    Kl = lax.dynamic_slice_in_dim(K_ext, my * HQ_LOCAL, HQ_LOCAL, axis=2)
    Vl = lax.dynamic_slice_in_dim(V_ext, my * HQ_LOCAL, HQ_LOCAL, axis=2)
    Kl = jnp.transpose(Kl, (0, 2, 1, 3))
    Vl = jnp.transpose(Vl, (0, 2, 1, 3))

    def body(x_ref, wq_ref, k_ref, v_ref, wo_ref, out_ref,
             comm_ref, send_sems, recv_sems):
        me = lax.axis_index("i")
        partners = (me ^ 1, me ^ 3, me ^ 4)

        bar = pltpu.get_barrier_semaphore()
        for p in partners:
            pl.semaphore_signal(bar, inc=1, device_id=(p,),
                                device_id_type=pl.DeviceIdType.MESH)
        pl.semaphore_wait(bar, 3)

        qb = lax.broadcasted_iota(jnp.int32, (SQ, SKV), 0) // 64
        kb = lax.broadcasted_iota(jnp.int32, (SQ, SKV), 1) // 64
        mask = (qb == kb) | ((kb % 4) == (qb % 4))
        for b in range(B):
            xb = x_ref[b]
            Qb = jnp.dot(xb, wq_ref[...], preferred_element_type=jnp.float32)
            ctxs = []
            for hl in range(HQ_LOCAL):
                q = Qb[:, hl * DH:(hl + 1) * DH]
                k = k_ref[b, hl]
                v = v_ref[b, hl]
                s = lax.dot_general(
                    q, k, (((1,), (1,)), ((), ())),
                    preferred_element_type=jnp.float32) * SCALE
                s = jnp.where(mask, s, -1e9)
                m = jnp.max(s, axis=-1, keepdims=True)
                w = jnp.exp(s - m)
                w = w / jnp.sum(w, axis=-1, keepdims=True)
                ctxs.append(jnp.dot(w, v, preferred_element_type=jnp.float32))
            ctx = jnp.concatenate(ctxs, axis=1)
            out_ref[b] = jnp.dot(ctx, wo_ref[...],
                                 preferred_element_type=jnp.float32)

        for st in range(3):
            rdma = pltpu.make_async_remote_copy(
                src_ref=out_ref,
                dst_ref=comm_ref.at[st],
                send_sem=send_sems.at[st],
                recv_sem=recv_sems.at[st],
                device_id=(partners[st],),
                device_id_type=pl.DeviceIdType.MESH,
            )
            rdma.start()
            rdma.wait()
            out_ref[...] += comm_ref[st]

        @functools.partial(pl.run_scoped, sem2=pltpu.SemaphoreType.REGULAR)
        def _(sem2):
            for p in partners:
                pl.semaphore_signal(sem2, inc=1, device_id=(p,),
                                    device_id_type=pl.DeviceIdType.MESH)
            pl.semaphore_wait(sem2, 3)

    return pl.pallas_call(
        body,
        out_shape=jax.ShapeDtypeStruct((B, SQ, DM), jnp.float32),
        in_specs=[pl.BlockSpec(memory_space=pltpu.VMEM)] * 5,
        out_specs=pl.BlockSpec(memory_space=pltpu.VMEM),
        scratch_shapes=[
            pltpu.VMEM((3, B, SQ, DM), jnp.float32),
            pltpu.SemaphoreType.DMA((3,)),
            pltpu.SemaphoreType.DMA((3,)),
        ],
        compiler_params=pltpu.CompilerParams(collective_id=0),
    )(x, Wq, Kl, Vl, Wo)


# device time: 29444 ns/iter; 1.6793x vs baseline; 1.6793x over previous
import functools

import jax
import jax.numpy as jnp
from jax import lax
from jax.experimental import pallas as pl
from jax.experimental.pallas import tpu as pltpu

N_DEV = 8
B = 2
SQ = 256
SKV = 256
HQ_LOCAL = 4
DH = 64
DM = 512
SCALE = 0.125

STAGES = 3
CHUNKS = 8
CHUNK_ROWS = B * SQ // CHUNKS
CPB = CHUNKS // B


def kernel(x, Wq, K_ext, V_ext, Wo):
    my = lax.axis_index("i")
    Kl = lax.dynamic_slice_in_dim(K_ext, my * HQ_LOCAL, HQ_LOCAL, axis=2)
    Vl = lax.dynamic_slice_in_dim(V_ext, my * HQ_LOCAL, HQ_LOCAL, axis=2)
    Kl = jnp.transpose(Kl, (0, 2, 1, 3))
    Vl = jnp.transpose(Vl, (0, 2, 1, 3))

    def body(x_ref, wq_ref, k_ref, v_ref, wo_ref, out_ref,
             acc_ref, comm_ref, send_sems, recv_sems):
        me = lax.axis_index("i")
        partners = (me ^ 1, me ^ 3, me ^ 4)

        bar = pltpu.get_barrier_semaphore()
        for p in partners:
            pl.semaphore_signal(bar, inc=1, device_id=(p,),
                                device_id_type=pl.DeviceIdType.MESH)
        pl.semaphore_wait(bar, 3)

        qb = lax.broadcasted_iota(jnp.int32, (SQ, SKV), 0) // 64
        kb = lax.broadcasted_iota(jnp.int32, (SQ, SKV), 1) // 64
        mask = (qb == kb) | ((kb % 4) == (qb % 4))

        def compute_batch(b):
            xb = x_ref[b]
            Qb = jnp.dot(xb, wq_ref[...], preferred_element_type=jnp.float32)
            ctxs = []
            for hl in range(HQ_LOCAL):
                q = Qb[:, hl * DH:(hl + 1) * DH]
                k = k_ref[b, hl]
                v = v_ref[b, hl]
                s = lax.dot_general(
                    q, k, (((1,), (1,)), ((), ())),
                    preferred_element_type=jnp.float32) * SCALE
                s = jnp.where(mask, s, -1e9)
                m = jnp.max(s, axis=-1, keepdims=True)
                w = jnp.exp(s - m)
                w = w / jnp.sum(w, axis=-1, keepdims=True)
                ctxs.append(jnp.dot(w, v, preferred_element_type=jnp.float32))
            ctx = jnp.concatenate(ctxs, axis=1)
            o_b = jnp.dot(ctx, wo_ref[...],
                          preferred_element_type=jnp.float32)
            acc_ref[b * CPB:(b + 1) * CPB] = o_b.reshape(CPB, CHUNK_ROWS, DM)

        def rdma(s, c):
            k = s * CHUNKS + c
            return pltpu.make_async_remote_copy(
                src_ref=acc_ref.at[c],
                dst_ref=comm_ref.at[k],
                send_sem=send_sems.at[k],
                recv_sem=recv_sems.at[k],
                device_id=(partners[s],),
                device_id_type=pl.DeviceIdType.MESH,
            )

        compute_batch(0)
        for c in range(CPB):
            rdma(0, c).start()
        compute_batch(1)
        for c in range(CPB, CHUNKS):
            rdma(0, c).start()

        sched = sorted(
            ((s, c) for s in range(STAGES) for c in range(CHUNKS)),
            key=lambda sc: (sc[1] + 2 * sc[0], -sc[0]),
        )
        for s, c in sched:
            rdma(s, c).wait()
            acc_ref[c] += comm_ref[s * CHUNKS + c]
            if s + 1 < STAGES:
                rdma(s + 1, c).start()

        out_ref[0] = acc_ref[0:CPB].reshape(SQ, DM)
        out_ref[1] = acc_ref[CPB:CHUNKS].reshape(SQ, DM)

        @functools.partial(pl.run_scoped, sem2=pltpu.SemaphoreType.REGULAR)
        def _(sem2):
            for p in partners:
                pl.semaphore_signal(sem2, inc=1, device_id=(p,),
                                    device_id_type=pl.DeviceIdType.MESH)
            pl.semaphore_wait(sem2, 3)

    return pl.pallas_call(
        body,
        out_shape=jax.ShapeDtypeStruct((B, SQ, DM), jnp.float32),
        in_specs=[pl.BlockSpec(memory_space=pltpu.VMEM)] * 5,
        out_specs=pl.BlockSpec(memory_space=pltpu.VMEM),
        scratch_shapes=[
            pltpu.VMEM((CHUNKS, CHUNK_ROWS, DM), jnp.float32),
            pltpu.VMEM((STAGES * CHUNKS, CHUNK_ROWS, DM), jnp.float32),
            pltpu.SemaphoreType.DMA((STAGES * CHUNKS,)),
            pltpu.SemaphoreType.DMA((STAGES * CHUNKS,)),
        ],
        compiler_params=pltpu.CompilerParams(collective_id=0),
    )(x, Wq, Kl, Vl, Wo)
